# baseline (speedup 1.0000x reference)
import functools

import jax
import jax.numpy as jnp
import numpy as np
from jax import lax
from jax.experimental import pallas as pl
from jax.experimental.pallas import tpu as pltpu
from jax.experimental.pallas import tpu_sc as plsc

N = 10000
R = 8
H = 128
E = 320000
NRP = 81920
NP = 10240
NC = 2
NS = 16
NW = NC * NS
SUBK = 128
NSUB = 10
SUP = SUBK * NSUB
NSUP = E // SUP

_mesh = plsc.VectorSubcoreMesh(core_axis_name="c", subcore_axis_name="s")

_SEG_PER_TILE = NRP // NS
_ROWS_PER_TILE = NP // NS


def _count_body(dst_h, rel_h, cnt_out, dv, rv, seg2, ones_v, zb, sem, sem2,
                cnt_sh):
    c = lax.axis_index("c")
    s = lax.axis_index("s")
    wid = s * NC + c

    def f_ones(i, _):
        ones_v[pl.ds(i * 16, 16)] = jnp.ones((16,), jnp.int32)
        return 0
    lax.fori_loop(0, 8, f_ones, 0)

    def f_zero(i, _):
        zb[pl.ds(i * 16, 16)] = jnp.zeros((16,), jnp.int32)
        return 0
    lax.fori_loop(0, _SEG_PER_TILE // 16, f_zero, 0)

    pltpu.sync_copy(zb, cnt_sh.at[pl.ds(s * _SEG_PER_TILE, _SEG_PER_TILE)])
    plsc.subcore_barrier()

    nsup = (NSUP - wid + NW - 1) // NW

    base0 = wid * SUP
    pltpu.async_copy(dst_h.at[pl.ds(base0, SUP)], dv, sem)
    pltpu.async_copy(rel_h.at[pl.ds(base0, SUP)], rv, sem)

    def chunk(t, _):
        base = (wid + t * NW) * SUP
        pltpu.make_async_copy(dst_h.at[pl.ds(base, SUP)], dv, sem).wait()
        pltpu.make_async_copy(rel_h.at[pl.ds(base, SUP)], rv, sem).wait()

        def grp(g, _):
            def vec(j, _):
                o = g * 128 + j * 16
                seg2[g, pl.ds(j * 16, 16)] = (
                    dv[pl.ds(o, 16)] * R + rv[pl.ds(o, 16)])
                return 0
            lax.fori_loop(0, 8, vec, 0)
            return 0
        lax.fori_loop(0, NSUB, grp, 0)

        @pl.when(t + 1 < nsup)
        def _prefetch():
            nbase = (wid + (t + 1) * NW) * SUP
            pltpu.async_copy(dst_h.at[pl.ds(nbase, SUP)], dv, sem)
            pltpu.async_copy(rel_h.at[pl.ds(nbase, SUP)], rv, sem)

        descs = [pltpu.async_copy(ones_v, cnt_sh.at[seg2.at[g]], sem2,
                                  add=True)
                 for g in range(NSUB)]
        for d in descs:
            d.wait()
        return 0
    lax.fori_loop(0, nsup, chunk, 0)

    plsc.subcore_barrier()
    pltpu.sync_copy(
        cnt_sh.at[pl.ds(s * _SEG_PER_TILE, _SEG_PER_TILE)],
        cnt_out.at[pl.ds(c * NRP + s * _SEG_PER_TILE, _SEG_PER_TILE)])


_count = pl.kernel(
    _count_body,
    out_type=jax.ShapeDtypeStruct((NC * NRP,), jnp.int32),
    mesh=_mesh,
    scratch_types=[
        pltpu.VMEM((SUP,), jnp.int32),
        pltpu.VMEM((SUP,), jnp.int32),
        pltpu.VMEM((NSUB, 128), jnp.int32),
        pltpu.VMEM((128,), jnp.int32),
        pltpu.VMEM((_SEG_PER_TILE,), jnp.int32),
        pltpu.SemaphoreType.DMA,
        pltpu.SemaphoreType.DMA,
        pltpu.VMEM_SHARED((NRP,), jnp.int32),
    ],
)


def _agg_body(table_h, src_h, dst_h, rel_h, inv_h, apart,
              sv, dv, rv, gid2, seg2, dst2, rows_a, rows_b, ivals, zrow,
              sem_i, sem_ga, sem_gb, sem_sa, sem_sb, sem_e, a_sh):
    c = lax.axis_index("c")
    s = lax.axis_index("s")
    wid = s * NC + c

    def f_zrow(i, _):
        for hh in range(8):
            zrow[i, pl.ds(hh * 16, 16)] = jnp.zeros((16,), jnp.float32)
        return 0
    lax.fori_loop(0, 16, f_zrow, 0)

    def f_za(z, _):
        pltpu.sync_copy(zrow, a_sh.at[pl.ds(s * _ROWS_PER_TILE + z * 16, 16)])
        return 0
    lax.fori_loop(0, _ROWS_PER_TILE // 16, f_za, 0)
    plsc.subcore_barrier()

    nsup = (NSUP - wid + NW - 1) // NW
    bufs = [rows_a, rows_b]
    gsems = [sem_ga, sem_gb]
    ssems = [sem_sa, sem_sb]

    def scale(buf, sub):
        def srow(jg, _):
            iv16 = ivals[pl.ds(sub * 128 + jg * 16, 16)]
            for jj in range(16):
                splat_idx = jnp.full((16,), jj, jnp.int32)
                w = iv16.at[splat_idx].get(mode="promise_in_bounds")
                j = jg * 16 + jj
                for hh in range(8):
                    buf[j, pl.ds(hh * 16, 16)] = (
                        buf[j, pl.ds(hh * 16, 16)] * w)
            return 0
        lax.fori_loop(0, 8, srow, 0)

    base0 = wid * SUP
    pltpu.async_copy(src_h.at[pl.ds(base0, SUP)], sv, sem_e)
    pltpu.async_copy(dst_h.at[pl.ds(base0, SUP)], dv, sem_e)
    pltpu.async_copy(rel_h.at[pl.ds(base0, SUP)], rv, sem_e)

    def chunk(t, _):
        base = (wid + t * NW) * SUP
        pltpu.make_async_copy(src_h.at[pl.ds(base, SUP)], sv, sem_e).wait()
        pltpu.make_async_copy(dst_h.at[pl.ds(base, SUP)], dv, sem_e).wait()
        pltpu.make_async_copy(rel_h.at[pl.ds(base, SUP)], rv, sem_e).wait()

        def grp(g, _):
            def vec(j, _):
                o = g * 128 + j * 16
                s16 = sv[pl.ds(o, 16)]
                d16 = dv[pl.ds(o, 16)]
                r16 = rv[pl.ds(o, 16)]
                gid2[g, pl.ds(j * 16, 16)] = r16 * N + s16
                seg2[g, pl.ds(j * 16, 16)] = d16 * R + r16
                dst2[g, pl.ds(j * 16, 16)] = d16
                return 0
            lax.fori_loop(0, 8, vec, 0)
            return 0
        lax.fori_loop(0, NSUB, grp, 0)

        @pl.when(t + 1 < nsup)
        def _prefetch():
            nbase = (wid + (t + 1) * NW) * SUP
            pltpu.async_copy(src_h.at[pl.ds(nbase, SUP)], sv, sem_e)
            pltpu.async_copy(dst_h.at[pl.ds(nbase, SUP)], dv, sem_e)
            pltpu.async_copy(rel_h.at[pl.ds(nbase, SUP)], rv, sem_e)

        inv_descs = [
            pltpu.async_copy(inv_h.at[seg2.at[g]],
                             ivals.at[pl.ds(g * 128, 128)], sem_i)
            for g in range(NSUB)]

        g_descs = {}
        s_descs = {}
        g_descs[0] = pltpu.async_copy(table_h.at[gid2.at[0]], rows_a, sem_ga)
        for sub in range(NSUB):
            buf = bufs[sub % 2]
            if sub < NSUB - 1:
                if sub >= 1:
                    s_descs[sub - 1].wait()
                g_descs[sub + 1] = pltpu.async_copy(
                    table_h.at[gid2.at[sub + 1]], bufs[(sub + 1) % 2],
                    gsems[(sub + 1) % 2])
            g_descs[sub].wait()
            inv_descs[sub].wait()
            scale(buf, sub)
            s_descs[sub] = pltpu.async_copy(
                buf, a_sh.at[dst2.at[sub]], ssems[sub % 2], add=True)
        s_descs[NSUB - 2].wait()
        s_descs[NSUB - 1].wait()
        return 0
    lax.fori_loop(0, nsup, chunk, 0)

    plsc.subcore_barrier()
    pltpu.sync_copy(a_sh.at[pl.ds(s * _ROWS_PER_TILE, _ROWS_PER_TILE)],
                    apart.at[c, pl.ds(s * _ROWS_PER_TILE, _ROWS_PER_TILE)])


_agg = pl.kernel(
    _agg_body,
    out_type=jax.ShapeDtypeStruct((NC, NP, H), jnp.float32),
    mesh=_mesh,
    scratch_types=[
        pltpu.VMEM((SUP,), jnp.int32),
        pltpu.VMEM((SUP,), jnp.int32),
        pltpu.VMEM((SUP,), jnp.int32),
        pltpu.VMEM((NSUB, 128), jnp.int32),
        pltpu.VMEM((NSUB, 128), jnp.int32),
        pltpu.VMEM((NSUB, 128), jnp.int32),
        pltpu.VMEM((SUBK, H), jnp.float32),
        pltpu.VMEM((SUBK, H), jnp.float32),
        pltpu.VMEM((SUP,), jnp.float32),
        pltpu.VMEM((16, H), jnp.float32),
        pltpu.SemaphoreType.DMA,
        pltpu.SemaphoreType.DMA,
        pltpu.SemaphoreType.DMA,
        pltpu.SemaphoreType.DMA,
        pltpu.SemaphoreType.DMA,
        pltpu.SemaphoreType.DMA,
        pltpu.VMEM_SHARED((NP, H), jnp.float32),
    ],
)


_BN = 1000


def _t1_body(comb_ref, b_ref, cnt_ref, out_ref, inv_ref):
    r = pl.program_id(0)
    w0 = comb_ref[r, 0]
    w1 = comb_ref[r, 1]
    out_ref[0] = w0 * b_ref[0] + w1 * b_ref[1]
    csum = (cnt_ref[0] + cnt_ref[1]).astype(jnp.float32)
    inv_ref[...] = 1.0 / jnp.maximum(csum, 1.0)


def _t1(comb1, basis1, cnt_part):
    return pl.pallas_call(
        _t1_body,
        grid=(R, N // _BN),
        in_specs=[
            pl.BlockSpec(memory_space=pltpu.SMEM),
            pl.BlockSpec((2, _BN, H), lambda r, i: (0, i, 0)),
            pl.BlockSpec((NC, NRP // 128, 128), lambda r, i: (0, 0, 0)),
        ],
        out_specs=[
            pl.BlockSpec((1, _BN, H), lambda r, i: (r, i, 0)),
            pl.BlockSpec((NRP // 128, 128), lambda r, i: (0, 0)),
        ],
        out_shape=[
            jax.ShapeDtypeStruct((R, N, H), jnp.float32),
            jax.ShapeDtypeStruct((NRP // 128, 128), jnp.float32),
        ],
    )(comb1, basis1, cnt_part.reshape(NC, NRP // 128, 128))


def _t23_body(a_ref, root_ref, bias_ref, b2_ref, comb_ref, root2_ref,
              t2_ref, hr2_ref):
    hb = jnp.maximum(
        a_ref[0] + a_ref[1] + root_ref[...] + bias_ref[...], 0.0)
    m0 = jnp.dot(hb, b2_ref[0], preferred_element_type=jnp.float32)
    m1 = jnp.dot(hb, b2_ref[1], preferred_element_type=jnp.float32)
    hr2_ref[...] = jnp.dot(hb, root2_ref[...],
                           preferred_element_type=jnp.float32)
    for r in range(R):
        t2_ref[r] = comb_ref[r, 0] * m0 + comb_ref[r, 1] * m1


def _t23(apart, root1, bias1, basis2, comb2, root2):
    return pl.pallas_call(
        _t23_body,
        grid=(N // _BN,),
        in_specs=[
            pl.BlockSpec((NC, _BN, H), lambda i: (0, i, 0)),
            pl.BlockSpec((_BN, H), lambda i: (i, 0)),
            pl.BlockSpec((1, H), lambda i: (0, 0)),
            pl.BlockSpec((2, H, H), lambda i: (0, 0, 0)),
            pl.BlockSpec(memory_space=pltpu.SMEM),
            pl.BlockSpec((H, H), lambda i: (0, 0)),
        ],
        out_specs=[
            pl.BlockSpec((R, _BN, H), lambda i: (0, i, 0)),
            pl.BlockSpec((_BN, H), lambda i: (i, 0)),
        ],
        out_shape=[
            jax.ShapeDtypeStruct((R, N, H), jnp.float32),
            jax.ShapeDtypeStruct((N, H), jnp.float32),
        ],
    )(apart, root1, bias1.reshape(1, H), basis2, comb2, root2)


def _t4_body(b_ref, hr2_ref, bias_ref, out_ref):
    out_ref[...] = b_ref[0] + b_ref[1] + hr2_ref[...] + bias_ref[...]


def _t4(bpart, hr2, bias2):
    return pl.pallas_call(
        _t4_body,
        grid=(N // _BN,),
        in_specs=[
            pl.BlockSpec((NC, _BN, H), lambda i: (0, i, 0)),
            pl.BlockSpec((_BN, H), lambda i: (i, 0)),
            pl.BlockSpec((1, H), lambda i: (0, 0)),
        ],
        out_specs=pl.BlockSpec((_BN, H), lambda i: (i, 0)),
        out_shape=jax.ShapeDtypeStruct((N, H), jnp.float32),
    )(bpart, hr2, bias2.reshape(1, H))


def kernel(x, edge_index, edge_type, basis1, comb1, root1, bias1,
           basis2, comb2, root2, bias2):
    src = edge_index[0]
    dst = edge_index[1]
    rel = edge_type

    cnt_part = _count(dst, rel)
    table1, inv = _t1(comb1, basis1, cnt_part)
    table1 = table1.reshape(R * N, H)
    inv = inv.reshape(NRP)
    apart = _agg(table1, src, dst, rel, inv)
    table2, hr2 = _t23(apart, root1, bias1, basis2, comb2, root2)
    bpart = _agg(table2.reshape(R * N, H), src, dst, rel, inv)
    return _t4(bpart, hr2, bias2)


# speedup vs baseline: 1.0248x; 1.0248x over previous
import functools

import jax
import jax.numpy as jnp
import numpy as np
from jax import lax
from jax.experimental import pallas as pl
from jax.experimental.pallas import tpu as pltpu
from jax.experimental.pallas import tpu_sc as plsc

N = 10000
R = 8
H = 128
E = 320000
NRP = 81920
NP = 10240
NC = 2
NS = 16
NW = NC * NS
SUBK = 128
NSUB = 10
SUP = SUBK * NSUB
NSUP = E // SUP

_mesh = plsc.VectorSubcoreMesh(core_axis_name="c", subcore_axis_name="s")

_SEG_PER_TILE = NRP // NS
_ROWS_PER_TILE = NP // NS


def _count_body(dst_h, rel_h, cnt_out, dv, rv, seg2, ones_v, zb, sem, sem2,
                cnt_sh):
    c = lax.axis_index("c")
    s = lax.axis_index("s")
    wid = s * NC + c

    def f_ones(i, _):
        ones_v[pl.ds(i * 16, 16)] = jnp.ones((16,), jnp.int32)
        return 0
    lax.fori_loop(0, 8, f_ones, 0)

    def f_zero(i, _):
        zb[pl.ds(i * 16, 16)] = jnp.zeros((16,), jnp.int32)
        return 0
    lax.fori_loop(0, _SEG_PER_TILE // 16, f_zero, 0)

    pltpu.sync_copy(zb, cnt_sh.at[pl.ds(s * _SEG_PER_TILE, _SEG_PER_TILE)])
    plsc.subcore_barrier()

    nsup = (NSUP - wid + NW - 1) // NW

    base0 = wid * SUP
    pltpu.async_copy(dst_h.at[pl.ds(base0, SUP)], dv, sem)
    pltpu.async_copy(rel_h.at[pl.ds(base0, SUP)], rv, sem)

    def chunk(t, _):
        base = (wid + t * NW) * SUP
        pltpu.make_async_copy(dst_h.at[pl.ds(base, SUP)], dv, sem).wait()
        pltpu.make_async_copy(rel_h.at[pl.ds(base, SUP)], rv, sem).wait()

        def grp(g, _):
            def vec(j, _):
                o = g * 128 + j * 16
                seg2[g, pl.ds(j * 16, 16)] = (
                    dv[pl.ds(o, 16)] * R + rv[pl.ds(o, 16)])
                return 0
            lax.fori_loop(0, 8, vec, 0)
            return 0
        lax.fori_loop(0, NSUB, grp, 0)

        @pl.when(t + 1 < nsup)
        def _prefetch():
            nbase = (wid + (t + 1) * NW) * SUP
            pltpu.async_copy(dst_h.at[pl.ds(nbase, SUP)], dv, sem)
            pltpu.async_copy(rel_h.at[pl.ds(nbase, SUP)], rv, sem)

        descs = [pltpu.async_copy(ones_v, cnt_sh.at[seg2.at[g]], sem2,
                                  add=True)
                 for g in range(NSUB)]
        for d in descs:
            d.wait()
        return 0
    lax.fori_loop(0, nsup, chunk, 0)

    plsc.subcore_barrier()
    pltpu.sync_copy(
        cnt_sh.at[pl.ds(s * _SEG_PER_TILE, _SEG_PER_TILE)],
        cnt_out.at[pl.ds(c * NRP + s * _SEG_PER_TILE, _SEG_PER_TILE)])


_count = pl.kernel(
    _count_body,
    out_type=jax.ShapeDtypeStruct((NC * NRP,), jnp.int32),
    mesh=_mesh,
    scratch_types=[
        pltpu.VMEM((SUP,), jnp.int32),
        pltpu.VMEM((SUP,), jnp.int32),
        pltpu.VMEM((NSUB, 128), jnp.int32),
        pltpu.VMEM((128,), jnp.int32),
        pltpu.VMEM((_SEG_PER_TILE,), jnp.int32),
        pltpu.SemaphoreType.DMA,
        pltpu.SemaphoreType.DMA,
        pltpu.VMEM_SHARED((NRP,), jnp.int32),
    ],
)


def _agg_body(table_h, src_h, dst_h, rel_h, inv_h, apart,
              sv, dv, rv, gid2, seg2, dst2, rows_a, rows_b, ivals, zrow,
              sem_i, sem_ga, sem_gb, sem_sa, sem_sb, sem_e, a_sh):
    c = lax.axis_index("c")
    s = lax.axis_index("s")
    wid = s * NC + c

    def f_zrow(i, _):
        for hh in range(8):
            zrow[i, pl.ds(hh * 16, 16)] = jnp.zeros((16,), jnp.float32)
        return 0
    lax.fori_loop(0, 16, f_zrow, 0)

    def f_za(z, _):
        pltpu.sync_copy(zrow, a_sh.at[pl.ds(s * _ROWS_PER_TILE + z * 16, 16)])
        return 0
    lax.fori_loop(0, _ROWS_PER_TILE // 16, f_za, 0)
    plsc.subcore_barrier()

    nsup = (NSUP - wid + NW - 1) // NW
    bufs = [rows_a, rows_b]
    gsems = [sem_ga, sem_gb]
    ssems = [sem_sa, sem_sb]

    def scale(buf, sub):
        def srow(jg, _):
            iv16 = ivals[pl.ds(sub * 128 + jg * 16, 16)]
            for jj in range(16):
                splat_idx = jnp.full((16,), jj, jnp.int32)
                w = iv16.at[splat_idx].get(mode="promise_in_bounds")
                j = jg * 16 + jj
                for hh in range(8):
                    buf[j, pl.ds(hh * 16, 16)] = (
                        buf[j, pl.ds(hh * 16, 16)] * w)
            return 0
        lax.fori_loop(0, 8, srow, 0)

    base0 = wid * SUP
    pltpu.async_copy(src_h.at[pl.ds(base0, SUP)], sv, sem_e)
    pltpu.async_copy(dst_h.at[pl.ds(base0, SUP)], dv, sem_e)
    pltpu.async_copy(rel_h.at[pl.ds(base0, SUP)], rv, sem_e)

    def chunk(t, _):
        base = (wid + t * NW) * SUP
        pltpu.make_async_copy(src_h.at[pl.ds(base, SUP)], sv, sem_e).wait()
        pltpu.make_async_copy(dst_h.at[pl.ds(base, SUP)], dv, sem_e).wait()
        pltpu.make_async_copy(rel_h.at[pl.ds(base, SUP)], rv, sem_e).wait()

        def grp(g, _):
            def vec(j, _):
                o = g * 128 + j * 16
                s16 = sv[pl.ds(o, 16)]
                d16 = dv[pl.ds(o, 16)]
                r16 = rv[pl.ds(o, 16)]
                gid2[g, pl.ds(j * 16, 16)] = r16 * N + s16
                seg2[g, pl.ds(j * 16, 16)] = d16 * R + r16
                dst2[g, pl.ds(j * 16, 16)] = d16
                return 0
            lax.fori_loop(0, 8, vec, 0)
            return 0
        lax.fori_loop(0, NSUB, grp, 0)

        @pl.when(t + 1 < nsup)
        def _prefetch():
            nbase = (wid + (t + 1) * NW) * SUP
            pltpu.async_copy(src_h.at[pl.ds(nbase, SUP)], sv, sem_e)
            pltpu.async_copy(dst_h.at[pl.ds(nbase, SUP)], dv, sem_e)
            pltpu.async_copy(rel_h.at[pl.ds(nbase, SUP)], rv, sem_e)

        inv_descs = [
            pltpu.async_copy(inv_h.at[seg2.at[g]],
                             ivals.at[pl.ds(g * 128, 128)], sem_i)
            for g in range(NSUB)]

        g_descs = {}
        s_descs = {}
        g_descs[0] = pltpu.async_copy(table_h.at[gid2.at[0]], rows_a, sem_ga)
        for sub in range(NSUB):
            buf = bufs[sub % 2]
            if sub < NSUB - 1:
                if sub >= 1:
                    s_descs[sub - 1].wait()
                g_descs[sub + 1] = pltpu.async_copy(
                    table_h.at[gid2.at[sub + 1]], bufs[(sub + 1) % 2],
                    gsems[(sub + 1) % 2])
            g_descs[sub].wait()
            inv_descs[sub].wait()
            scale(buf, sub)
            s_descs[sub] = pltpu.async_copy(
                buf, a_sh.at[dst2.at[sub]], ssems[sub % 2], add=True)
        s_descs[NSUB - 2].wait()
        s_descs[NSUB - 1].wait()
        return 0
    lax.fori_loop(0, nsup, chunk, 0)

    plsc.subcore_barrier()
    pltpu.sync_copy(a_sh.at[pl.ds(s * _ROWS_PER_TILE, _ROWS_PER_TILE)],
                    apart.at[c, pl.ds(s * _ROWS_PER_TILE, _ROWS_PER_TILE)])


_agg = pl.kernel(
    _agg_body,
    out_type=jax.ShapeDtypeStruct((NC, NP, H), jnp.float32),
    mesh=_mesh,
    scratch_types=[
        pltpu.VMEM((SUP,), jnp.int32),
        pltpu.VMEM((SUP,), jnp.int32),
        pltpu.VMEM((SUP,), jnp.int32),
        pltpu.VMEM((NSUB, 128), jnp.int32),
        pltpu.VMEM((NSUB, 128), jnp.int32),
        pltpu.VMEM((NSUB, 128), jnp.int32),
        pltpu.VMEM((SUBK, H), jnp.float32),
        pltpu.VMEM((SUBK, H), jnp.float32),
        pltpu.VMEM((SUP,), jnp.float32),
        pltpu.VMEM((16, H), jnp.float32),
        pltpu.SemaphoreType.DMA,
        pltpu.SemaphoreType.DMA,
        pltpu.SemaphoreType.DMA,
        pltpu.SemaphoreType.DMA,
        pltpu.SemaphoreType.DMA,
        pltpu.SemaphoreType.DMA,
        pltpu.VMEM_SHARED((NP, H), jnp.float32),
    ],
)


_BN = 1000


def _t0_body(cnt_ref, inv_ref):
    csum = (cnt_ref[0] + cnt_ref[1]).astype(jnp.float32)
    inv_ref[...] = 1.0 / jnp.maximum(csum, 1.0)


def _t0(cnt_part):
    return pl.pallas_call(
        _t0_body,
        out_shape=jax.ShapeDtypeStruct((NRP // 128, 128), jnp.float32),
    )(cnt_part.reshape(NC, NRP // 128, 128))


def _t1_body(comb_ref, b_ref, out_ref):
    r = pl.program_id(0)
    w0 = comb_ref[r, 0]
    w1 = comb_ref[r, 1]
    out_ref[0] = w0 * b_ref[0] + w1 * b_ref[1]


def _t1(comb1, basis1):
    return pl.pallas_call(
        _t1_body,
        grid=(R, N // _BN),
        in_specs=[
            pl.BlockSpec(memory_space=pltpu.SMEM),
            pl.BlockSpec((2, _BN, H), lambda r, i: (0, i, 0)),
        ],
        out_specs=pl.BlockSpec((1, _BN, H), lambda r, i: (r, i, 0)),
        out_shape=jax.ShapeDtypeStruct((R, N, H), jnp.float32),
    )(comb1, basis1)


def _t23_body(a_ref, root_ref, bias_ref, b2_ref, comb_ref, root2_ref,
              t2_ref, hr2_ref):
    hb = jnp.maximum(
        a_ref[0] + a_ref[1] + root_ref[...] + bias_ref[...], 0.0)
    m0 = jnp.dot(hb, b2_ref[0], preferred_element_type=jnp.float32)
    m1 = jnp.dot(hb, b2_ref[1], preferred_element_type=jnp.float32)
    hr2_ref[...] = jnp.dot(hb, root2_ref[...],
                           preferred_element_type=jnp.float32)
    for r in range(R):
        t2_ref[r] = comb_ref[r, 0] * m0 + comb_ref[r, 1] * m1


def _t23(apart, root1, bias1, basis2, comb2, root2):
    return pl.pallas_call(
        _t23_body,
        grid=(N // _BN,),
        in_specs=[
            pl.BlockSpec((NC, _BN, H), lambda i: (0, i, 0)),
            pl.BlockSpec((_BN, H), lambda i: (i, 0)),
            pl.BlockSpec((1, H), lambda i: (0, 0)),
            pl.BlockSpec((2, H, H), lambda i: (0, 0, 0)),
            pl.BlockSpec(memory_space=pltpu.SMEM),
            pl.BlockSpec((H, H), lambda i: (0, 0)),
        ],
        out_specs=[
            pl.BlockSpec((R, _BN, H), lambda i: (0, i, 0)),
            pl.BlockSpec((_BN, H), lambda i: (i, 0)),
        ],
        out_shape=[
            jax.ShapeDtypeStruct((R, N, H), jnp.float32),
            jax.ShapeDtypeStruct((N, H), jnp.float32),
        ],
    )(apart, root1, bias1.reshape(1, H), basis2, comb2, root2)


def _t4_body(b_ref, hr2_ref, bias_ref, out_ref):
    out_ref[...] = b_ref[0] + b_ref[1] + hr2_ref[...] + bias_ref[...]


def _t4(bpart, hr2, bias2):
    return pl.pallas_call(
        _t4_body,
        grid=(N // _BN,),
        in_specs=[
            pl.BlockSpec((NC, _BN, H), lambda i: (0, i, 0)),
            pl.BlockSpec((_BN, H), lambda i: (i, 0)),
            pl.BlockSpec((1, H), lambda i: (0, 0)),
        ],
        out_specs=pl.BlockSpec((_BN, H), lambda i: (i, 0)),
        out_shape=jax.ShapeDtypeStruct((N, H), jnp.float32),
    )(bpart, hr2, bias2.reshape(1, H))


def kernel(x, edge_index, edge_type, basis1, comb1, root1, bias1,
           basis2, comb2, root2, bias2):
    src = edge_index[0]
    dst = edge_index[1]
    rel = edge_type

    cnt_part = _count(dst, rel)
    inv = _t0(cnt_part).reshape(NRP)
    table1 = _t1(comb1, basis1).reshape(R * N, H)
    apart = _agg(table1, src, dst, rel, inv)
    table2, hr2 = _t23(apart, root1, bias1, basis2, comb2, root2)
    bpart = _agg(table2.reshape(R * N, H), src, dst, rel, inv)
    return _t4(bpart, hr2, bias2)
